# baseline (speedup 1.0000x reference)
import jax
import jax.numpy as jnp
from jax.experimental import pallas as pl
from jax.experimental.pallas import tpu as pltpu

_SLAB = 65536
_HALO = 4096
_SCALE = 1.0 / 32.0


def _grid_rev_kernel(slab_ref, fxc_ref, fys_ref, fzs_ref, myz_ref,
                     out_ref, tail_ref):
    i = pl.program_id(0)
    b = slab_ref.shape[0]
    slab = slab_ref[...].reshape(b, _SLAB)
    halo = jnp.where(i == 0, 0.0, tail_ref[...])
    w = jnp.concatenate([halo, slab], axis=1)
    tail_ref[...] = slab[:, _SLAB - _HALO:]

    def sh(a, k):
        return jnp.concatenate([jnp.zeros((b, k), jnp.float32), a[:, :-k]], axis=1)

    gz = w + sh(w, 1)
    gy = w + sh(w, 64)
    gzy = gz + sh(gz, 64)

    wsum = gzy[:, _HALO:] + gzy[:, :_SLAB]
    sx1 = gzy[:, _HALO:]
    sy1 = gz[:, _HALO:] + gz[:, :_SLAB]
    sz1 = gy[:, _HALO:] + gy[:, :_SLAB]

    m = myz_ref[0] * jnp.sign(wsum)
    rs = _SCALE / jnp.maximum(wsum, 1e-30)
    fx0 = fxc_ref[...] + (_SLAB // 4096) * i * _SCALE

    out_ref[0] = (fx0 + sx1 * rs) * m
    out_ref[1] = (fys_ref[...] + sy1 * rs) * m
    out_ref[2] = (fzs_ref[...] + sz1 * rs) * m


def kernel(grid):
    B = grid.shape[0]
    gv = grid.reshape(B, 4096, 64)

    l = jnp.arange(_SLAB, dtype=jnp.int32)[None, :]
    lz = l % 64
    ly = (l // 64) % 64
    lx = l // 4096
    fxc = (lx.astype(jnp.float32) - 33.0) * _SCALE
    fys = (ly.astype(jnp.float32) - 33.0) * _SCALE
    fzs = (lz.astype(jnp.float32) - 33.0) * _SCALE
    myz = ((ly >= 1) & (lz >= 1)).astype(jnp.float32)
    myz2 = jnp.stack([jnp.where(lx >= 1, myz, 0.0), myz], axis=0)

    const_spec = pl.BlockSpec((1, _SLAB), lambda i: (0, 0))
    out = pl.pallas_call(
        _grid_rev_kernel,
        grid=(64 * 4096 // _SLAB,),
        in_specs=[
            pl.BlockSpec((B, _SLAB // 64, 64), lambda i: (0, i, 0)),
            const_spec, const_spec, const_spec,
            pl.BlockSpec((1, 1, _SLAB), lambda i: (jnp.minimum(i, 1), 0, 0)),
        ],
        out_specs=pl.BlockSpec((3, B, _SLAB), lambda i: (0, 0, i)),
        out_shape=jax.ShapeDtypeStruct((3, B, 64 * 64 * 64), jnp.float32),
        scratch_shapes=[pltpu.VMEM((B, _HALO), jnp.float32)],
    )(gv, fxc, fys, fzs, myz2)
    return out.transpose(1, 2, 0)


# speedup vs baseline: 1.0333x; 1.0333x over previous
import jax
import jax.numpy as jnp
from jax.experimental import pallas as pl
from jax.experimental.pallas import tpu as pltpu

_SLAB = 32768
_HALO = 4096
_SCALE = 1.0 / 32.0


def _grid_rev_kernel(slab_ref, halo_ref, fxc_ref, fys_ref, fzs_ref, myz_ref,
                     out_ref):
    i = pl.program_id(0)
    b = slab_ref.shape[0]
    slab = slab_ref[...].reshape(b, _SLAB)
    halo = halo_ref[...].reshape(b, _HALO)
    w = jnp.concatenate([halo, slab], axis=1)

    def sh(a, k):
        return jnp.concatenate([jnp.zeros((b, k), jnp.float32), a[:, :-k]], axis=1)

    gz = w + sh(w, 1)
    gy = w + sh(w, 64)
    gzy = gz + sh(gz, 64)

    wsum = gzy[:, _HALO:] + gzy[:, :_SLAB]
    sx1 = gzy[:, _HALO:]
    sy1 = gz[:, _HALO:] + gz[:, :_SLAB]
    sz1 = gy[:, _HALO:] + gy[:, :_SLAB]

    m = myz_ref[0] * jnp.sign(wsum)
    rs = _SCALE / jnp.maximum(wsum, 1e-30)
    fx0 = fxc_ref[...] + (_SLAB // 4096) * i * _SCALE

    out_ref[0] = (fx0 + sx1 * rs) * m
    out_ref[1] = (fys_ref[...] + sy1 * rs) * m
    out_ref[2] = (fzs_ref[...] + sz1 * rs) * m


def kernel(grid):
    B = grid.shape[0]
    gv = grid.reshape(B, 4096, 64)

    l = jnp.arange(_SLAB, dtype=jnp.int32)[None, :]
    lz = l % 64
    ly = (l // 64) % 64
    lx = l // 4096
    fxc = (lx.astype(jnp.float32) - 33.0) * _SCALE
    fys = (ly.astype(jnp.float32) - 33.0) * _SCALE
    fzs = (lz.astype(jnp.float32) - 33.0) * _SCALE
    myz = ((ly >= 1) & (lz >= 1)).astype(jnp.float32)
    myz2 = jnp.stack([jnp.where(lx >= 1, myz, 0.0), myz], axis=0)

    const_spec = pl.BlockSpec((1, _SLAB), lambda i: (0, 0))
    out = pl.pallas_call(
        _grid_rev_kernel,
        grid=(64 * 4096 // _SLAB,),
        in_specs=[
            pl.BlockSpec((B, _SLAB // 64, 64), lambda i: (0, i, 0)),
            pl.BlockSpec((B, _HALO // 64, 64),
                         lambda i: (0, jnp.maximum((_SLAB // 4096) * i - 1, 0), 0)),
            const_spec, const_spec, const_spec,
            pl.BlockSpec((1, 1, _SLAB), lambda i: (jnp.minimum(i, 1), 0, 0)),
        ],
        out_specs=pl.BlockSpec((3, B, _SLAB), lambda i: (0, 0, i)),
        out_shape=jax.ShapeDtypeStruct((3, B, 64 * 64 * 64), jnp.float32),
    )(gv, gv, fxc, fys, fzs, myz2)
    return out.transpose(1, 2, 0)


# speedup vs baseline: 1.0828x; 1.0480x over previous
import jax
import jax.numpy as jnp
from jax.experimental import pallas as pl

_SLAB = 32768
_HALO = 4096


def _grid_rev_kernel(slab_ref, halo_ref, out_ref):
    i = pl.program_id(0)
    b = slab_ref.shape[0]
    halo = halo_ref[...].reshape(b, _HALO)
    slab = slab_ref[...].reshape(b, _SLAB)
    w = jnp.concatenate([halo, slab], axis=1)

    def sh(a, k):
        return jnp.concatenate([jnp.zeros((b, k), jnp.float32), a[:, :-k]], axis=1)

    gz = w + sh(w, 1)
    gy = w + sh(w, 64)
    gzy = gz + sh(gz, 64)

    wsum = gzy[:, _HALO:] + gzy[:, :_SLAB]
    sx1 = gzy[:, _HALO:]
    sy1 = gz[:, _HALO:] + gz[:, :_SLAB]
    sz1 = gy[:, _HALO:] + gy[:, :_SLAB]

    l = jax.lax.broadcasted_iota(jnp.int32, (b, _SLAB), 1)
    jz = l % 64
    jy = (l // 64) % 64
    jx = (l // 4096) + (_SLAB // 4096) * i

    interior = (jx >= 1) & (jy >= 1) & (jz >= 1)
    mask = interior & (wsum > 0.0)
    rs = (1.0 / 32.0) / jnp.where(mask, wsum, 1.0)
    fxs = (jx.astype(jnp.float32) - 33.0) * (1.0 / 32.0)
    fys = (jy.astype(jnp.float32) - 33.0) * (1.0 / 32.0)
    fzs = (jz.astype(jnp.float32) - 33.0) * (1.0 / 32.0)

    out_ref[0] = jnp.where(mask, fxs + sx1 * rs, 0.0)
    out_ref[1] = jnp.where(mask, fys + sy1 * rs, 0.0)
    out_ref[2] = jnp.where(mask, fzs + sz1 * rs, 0.0)


def kernel(grid):
    B = grid.shape[0]
    gv = grid.reshape(B, 4096, 64)
    out = pl.pallas_call(
        _grid_rev_kernel,
        grid=(64 * 4096 // _SLAB,),
        in_specs=[
            pl.BlockSpec((B, _SLAB // 64, 64), lambda i: (0, i, 0)),
            pl.BlockSpec((B, _HALO // 64, 64),
                         lambda i: (0, jnp.maximum((_SLAB // 4096) * i - 1, 0), 0)),
        ],
        out_specs=pl.BlockSpec((3, B, _SLAB), lambda i: (0, 0, i)),
        out_shape=jax.ShapeDtypeStruct((3, B, 64 * 64 * 64), jnp.float32),
    )(gv, gv)
    return out.transpose(1, 2, 0)


# speedup vs baseline: 1.2331x; 1.1388x over previous
import jax
import jax.numpy as jnp
from jax.experimental import pallas as pl

_SLAB = 32768
_HALO = 4096


def _grid_rev_kernel(slab_ref, halo_ref, out_ref):
    i = pl.program_id(0)
    b = slab_ref.shape[0]
    halo = halo_ref[...].astype(jnp.bfloat16).reshape(b, _HALO)
    slab = slab_ref[...].astype(jnp.bfloat16).reshape(b, _SLAB)
    w = jnp.concatenate([halo, slab], axis=1)

    def sh(a, k):
        return jnp.concatenate([jnp.zeros((b, k), jnp.bfloat16), a[:, :-k]], axis=1)

    gz = w + sh(w, 1)
    gy = w + sh(w, 64)
    gzy = gz + sh(gz, 64)

    wsum = (gzy[:, _HALO:] + gzy[:, :_SLAB]).astype(jnp.float32)
    sx1 = gzy[:, _HALO:].astype(jnp.float32)
    sy1 = (gz[:, _HALO:] + gz[:, :_SLAB]).astype(jnp.float32)
    sz1 = (gy[:, _HALO:] + gy[:, :_SLAB]).astype(jnp.float32)

    l = jax.lax.broadcasted_iota(jnp.int32, (b, _SLAB), 1)
    jz = l % 64
    jy = (l // 64) % 64
    jx = (l // 4096) + (_SLAB // 4096) * i

    interior = (jx >= 1) & (jy >= 1) & (jz >= 1)
    mask = interior & (wsum > 0.0)
    rs = (1.0 / 32.0) / jnp.where(mask, wsum, 1.0)
    fxs = (jx.astype(jnp.float32) - 33.0) * (1.0 / 32.0)
    fys = (jy.astype(jnp.float32) - 33.0) * (1.0 / 32.0)
    fzs = (jz.astype(jnp.float32) - 33.0) * (1.0 / 32.0)

    out_ref[0] = jnp.where(mask, fxs + sx1 * rs, 0.0)
    out_ref[1] = jnp.where(mask, fys + sy1 * rs, 0.0)
    out_ref[2] = jnp.where(mask, fzs + sz1 * rs, 0.0)


def kernel(grid):
    B = grid.shape[0]
    gv = grid.reshape(B, 4096, 64)
    out = pl.pallas_call(
        _grid_rev_kernel,
        grid=(64 * 4096 // _SLAB,),
        in_specs=[
            pl.BlockSpec((B, _SLAB // 64, 64), lambda i: (0, i, 0)),
            pl.BlockSpec((B, _HALO // 64, 64),
                         lambda i: (0, jnp.maximum((_SLAB // 4096) * i - 1, 0), 0)),
        ],
        out_specs=pl.BlockSpec((3, B, _SLAB), lambda i: (0, 0, i)),
        out_shape=jax.ShapeDtypeStruct((3, B, 64 * 64 * 64), jnp.float32),
    )(gv, gv)
    return out.transpose(1, 2, 0)
